# baseline (speedup 1.0000x reference)
import functools

import jax
import jax.numpy as jnp
from jax import lax
from jax.experimental import pallas as pl
from jax.experimental.pallas import tpu as pltpu
from jax.experimental.pallas import tpu_sc as plsc

_NUM_DOMAINS = 26
_BATCH = 16384
_NUM_CORES = 2
_NUM_SUBCORES = 16
_NW = _NUM_CORES * _NUM_SUBCORES
_PER_W = _BATCH // _NW
_LANES = 16


def _softplus_vec(w):
    a = jnp.abs(w)
    t = 1.0 + jnp.exp(-a)
    u = t - 1.0
    y = u * (6.0 + u) / (6.0 + 4.0 * u)
    for _ in range(2):
        y = y - 1.0 + t * jnp.exp(-y)
    return jnp.maximum(w, 0.0) + y


@functools.partial(
    pl.kernel,
    mesh=plsc.VectorSubcoreMesh(core_axis_name="c", subcore_axis_name="s"),
    out_type=jax.ShapeDtypeStruct((_BATCH,), jnp.float32),
    compiler_params=pltpu.CompilerParams(
        needs_layout_passes=False,
        disable_bounds_checks=True,
        skip_device_barrier=True,
        use_tc_tiling_on_sc=False,
    ),
    scratch_types=[
        pltpu.VMEM((_NUM_DOMAINS,), jnp.float32),
        pltpu.VMEM((2 * _LANES,), jnp.float32),
        pltpu.VMEM((_PER_W,), jnp.int32),
        pltpu.VMEM((_PER_W,), jnp.float32),
        pltpu.VMEM((_PER_W,), jnp.float32),
        pltpu.SemaphoreType.DMA,
        pltpu.SemaphoreType.DMA,
        pltpu.SemaphoreType.DMA,
    ],
)
def _domain_table_sc(w_hbm, idx_hbm, x_hbm, out_hbm,
                     w_v, tab_v, idx_v, x_v, out_v, sem_i, sem_x, sem_o):
    wid = lax.axis_index("s") * _NUM_CORES + lax.axis_index("c")
    base = wid * _PER_W

    idx_cp = pltpu.async_copy(idx_hbm.at[pl.ds(base, _PER_W)], idx_v, sem_i)
    x_cp = pltpu.async_copy(x_hbm.at[pl.ds(base, _PER_W)], x_v, sem_x)

    pltpu.sync_copy(w_hbm, w_v)
    lane = jax.lax.iota(jnp.int32, _LANES)
    w0 = w_v[pl.ds(0, _LANES)]
    w1 = plsc.load_gather(w_v, [jnp.minimum(lane + _LANES, _NUM_DOMAINS - 1)])
    sp0 = _softplus_vec(w0)
    sp1 = _softplus_vec(w1)
    sp1_valid = jnp.where(lane < (_NUM_DOMAINS - _LANES), sp1, 0.0)
    total = sp0 + sp1_valid
    for shift in (8, 4, 2, 1):
        tab_v[pl.ds(0, _LANES)] = total
        total = total + plsc.load_gather(tab_v, [lane ^ shift])
    scale = _NUM_DOMAINS / total
    tab_v[pl.ds(0, _LANES)] = sp0 * scale
    tab_v[pl.ds(_LANES, _LANES)] = sp1 * scale

    idx_cp.wait()
    x_cp.wait()

    out_cps = []
    chunk = _PER_W // 4
    for g in range(4):
        lo = g * chunk

        @plsc.parallel_loop(lo, lo + chunk, _LANES, unroll=8)
        def _gather_body(i):
            sl = pl.ds(i, _LANES)
            wv = plsc.load_gather(tab_v, [idx_v[sl]])
            out_v[sl] = x_v[sl] * wv

        out_cps.append(pltpu.async_copy(
            out_v.at[pl.ds(lo, chunk)],
            out_hbm.at[pl.ds(base + lo, chunk)], sem_o))
    for cp in out_cps:
        cp.wait()


def kernel(idxes, x, raw_weights):
    out = _domain_table_sc(raw_weights, idxes, x.reshape(_BATCH))
    return out.reshape(_BATCH, 1)


# speedup vs baseline: 1.0079x; 1.0079x over previous
import functools

import jax
import jax.numpy as jnp
from jax import lax
from jax.experimental import pallas as pl
from jax.experimental.pallas import tpu as pltpu
from jax.experimental.pallas import tpu_sc as plsc

_NUM_DOMAINS = 26
_BATCH = 16384
_NUM_CORES = 2
_NUM_SUBCORES = 16
_NW = _NUM_CORES * _NUM_SUBCORES
_PER_W = _BATCH // _NW
_LANES = 16


def _softplus_vec(w):
    a = jnp.abs(w)
    t = 1.0 + jnp.exp(-a)
    u = t - 1.0
    y = u * (6.0 + u) / (6.0 + 4.0 * u)
    y = y - 1.0 + t * jnp.exp(-y)
    return jnp.maximum(w, 0.0) + y


@functools.partial(
    pl.kernel,
    mesh=plsc.VectorSubcoreMesh(core_axis_name="c", subcore_axis_name="s"),
    out_type=jax.ShapeDtypeStruct((_BATCH,), jnp.float32),
    compiler_params=pltpu.CompilerParams(
        needs_layout_passes=False,
        disable_bounds_checks=True,
        skip_device_barrier=True,
        use_tc_tiling_on_sc=False,
    ),
    scratch_types=[
        pltpu.VMEM((_NUM_DOMAINS,), jnp.float32),
        pltpu.VMEM((2 * _LANES,), jnp.float32),
        pltpu.VMEM((_PER_W,), jnp.int32),
        pltpu.VMEM((_PER_W,), jnp.float32),
        pltpu.VMEM((_PER_W,), jnp.float32),
        pltpu.SemaphoreType.DMA,
        pltpu.SemaphoreType.DMA,
        pltpu.SemaphoreType.DMA,
    ],
)
def _domain_table_sc(w_hbm, idx_hbm, x_hbm, out_hbm,
                     w_v, tab_v, idx_v, x_v, out_v, sem_i, sem_x, sem_o):
    wid = lax.axis_index("s") * _NUM_CORES + lax.axis_index("c")
    base = wid * _PER_W

    idx_cp = pltpu.async_copy(idx_hbm.at[pl.ds(base, _PER_W)], idx_v, sem_i)
    x_cp = pltpu.async_copy(x_hbm.at[pl.ds(base, _PER_W)], x_v, sem_x)

    pltpu.sync_copy(w_hbm, w_v)
    lane = jax.lax.iota(jnp.int32, _LANES)
    w0 = w_v[pl.ds(0, _LANES)]
    w1 = plsc.load_gather(w_v, [jnp.minimum(lane + _LANES, _NUM_DOMAINS - 1)])
    sp0 = _softplus_vec(w0)
    sp1 = _softplus_vec(w1)
    sp1_valid = jnp.where(lane < (_NUM_DOMAINS - _LANES), sp1, 0.0)
    total = sp0 + sp1_valid
    for shift in (8, 4, 2, 1):
        tab_v[pl.ds(0, _LANES)] = total
        total = total + plsc.load_gather(tab_v, [lane ^ shift])
    scale = _NUM_DOMAINS / total
    tab_v[pl.ds(0, _LANES)] = sp0 * scale
    tab_v[pl.ds(_LANES, _LANES)] = sp1 * scale

    idx_cp.wait()
    x_cp.wait()

    @plsc.parallel_loop(0, _PER_W, _LANES, unroll=8)
    def _gather_body(i):
        sl = pl.ds(i, _LANES)
        wv = plsc.load_gather(tab_v, [idx_v[sl]])
        out_v[sl] = x_v[sl] * wv

    pltpu.sync_copy(out_v, out_hbm.at[pl.ds(base, _PER_W)])


def kernel(idxes, x, raw_weights):
    out = _domain_table_sc(raw_weights, idxes, x.reshape(_BATCH))
    return out.reshape(_BATCH, 1)
